# Initial kernel scaffold; baseline (speedup 1.0000x reference)
import jax
import jax.numpy as jnp
from jax.experimental import pallas as pl


def kernel(x, edge_index, batch, W1, W2, Wc):
    raise NotImplementedError("write your pallas kernel here")


# speedup vs baseline: 21.6859x; 21.6859x over previous
import functools

import jax
import jax.numpy as jnp
from jax import lax
from jax.experimental import pallas as pl
from jax.experimental.pallas import tpu as pltpu
from jax.experimental.pallas import tpu_sc as plsc

N_NODES = 10000
N_EDGES = 320000
D = 128
NUM_CLASSES = 10
NUM_GRAPHS = 128
B_EXP = 1.5
EPS = 1e-8

NC = 2
NS = 16
K = 80
C = N_EDGES // (NC * NS * K)
N_PAD = 10240
ROWS_PER_TILE = N_PAD // NS
DEG_W = 16

_mesh = plsc.VectorSubcoreMesh(core_axis_name="c", subcore_axis_name="s")


@functools.partial(
    pl.kernel,
    out_type=jax.ShapeDtypeStruct((NC, N_PAD, DEG_W), jnp.float32),
    mesh=_mesh,
    scratch_types=[
        pltpu.VMEM((C, K), jnp.int32),
        pltpu.VMEM((K, DEG_W), jnp.float32),
        pltpu.VMEM((128, DEG_W), jnp.float32),
        pltpu.VMEM_SHARED((N_PAD, DEG_W), jnp.float32),
    ],
)
def _deg_kernel(col_hbm, out_hbm, col_v, ones_v, stage, accum):
    c = lax.axis_index("c")
    s = lax.axis_index("s")
    zeros16 = jnp.zeros((16,), jnp.float32)
    ones16 = jnp.ones((16,), jnp.float32)

    def _zrow(i, _):
        stage[i, :] = zeros16
        return 0

    lax.fori_loop(0, 128, _zrow, 0)

    def _orow(i, _):
        ones_v[i, :] = ones16
        return 0

    lax.fori_loop(0, K, _orow, 0)

    base = s * ROWS_PER_TILE
    for t in range(5):
        pltpu.sync_copy(stage, accum.at[pl.ds(base + t * 128, 128)])
    plsc.subcore_barrier()

    pltpu.sync_copy(col_hbm.at[c, s], col_v)

    def _body(j, _):
        pltpu.sync_copy(ones_v, accum.at[col_v.at[j]], add=True)
        return 0

    lax.fori_loop(0, C, _body, 0)
    plsc.subcore_barrier()

    for t in range(5):
        pltpu.sync_copy(accum.at[pl.ds(base + t * 128, 128)], stage)
        pltpu.sync_copy(stage, out_hbm.at[c, pl.ds(base + t * 128, 128)])


@functools.partial(
    pl.kernel,
    out_type=jax.ShapeDtypeStruct((NC, N_PAD, D), jnp.float32),
    mesh=_mesh,
    scratch_types=[
        pltpu.VMEM((C * K,), jnp.int32),
        pltpu.VMEM((C, K), jnp.int32),
        pltpu.VMEM((2, K, D), jnp.float32),
        pltpu.VMEM_SHARED((N_PAD, D), jnp.float32),
        pltpu.SemaphoreType.DMA,
    ],
)
def _scatter_kernel(h_hbm, row_hbm, col_hbm, out_hbm,
                    row_v, col_v, msg, accum, sem):
    c = lax.axis_index("c")
    s = lax.axis_index("s")
    zeros16 = jnp.zeros((16,), jnp.float32)

    def _zrow(i, _):
        msg[0, i // 8, pl.ds((i % 8) * 16, 16)] = zeros16
        return 0

    lax.fori_loop(0, K * 8, _zrow, 0)

    base = s * ROWS_PER_TILE
    for t in range(ROWS_PER_TILE // K):
        pltpu.sync_copy(msg.at[0], accum.at[pl.ds(base + t * K, K)])
    plsc.subcore_barrier()

    pltpu.sync_copy(row_hbm.at[c, s], row_v)
    pltpu.sync_copy(col_hbm.at[c, s], col_v)

    def _gather_desc(j, slot):
        off = pl.multiple_of(j * K, K)
        return pltpu.make_async_copy(h_hbm.at[row_v.at[pl.ds(off, K)]],
                                     msg.at[slot], sem)

    def _gather(j, slot):
        _gather_desc(j, slot).start()

    _gather(0, 0)

    def _body(j, _):
        slot = lax.rem(j, 2)
        _gather_desc(j, slot).wait()

        @pl.when(j + 1 < C)
        def _():
            _gather(j + 1, 1 - slot)

        pltpu.sync_copy(msg.at[slot], accum.at[col_v.at[j]], add=True)
        return 0

    lax.fori_loop(0, C, _body, 0)
    plsc.subcore_barrier()

    for t in range(ROWS_PER_TILE // K):
        pltpu.sync_copy(accum.at[pl.ds(base + t * K, K)], msg.at[0])
        pltpu.sync_copy(msg.at[0], out_hbm.at[c, pl.ds(base + t * K, K)])


_RB = 1000


def _dis_block(deg_ref):
    deg = deg_ref[0, :, 0:1] + deg_ref[1, :, 0:1] + 1.0
    return lax.rsqrt(deg)


def _bcos_scale(h):
    nrm2 = jnp.sum(h * h, axis=1, keepdims=True)
    return lax.rsqrt(jnp.sqrt(nrm2) + EPS)


def _stage1_body(x_ref, w_ref, deg_ref, o_ref):
    x = x_ref[...]
    dis = _dis_block(deg_ref)
    h = x * _bcos_scale(x)
    o_ref[...] = jnp.dot(h, w_ref[...], preferred_element_type=jnp.float32,
                         precision=lax.Precision.HIGHEST) * dis


def _stage2_body(s_ref, hp_ref, deg_ref, w_ref, o_ref):
    dis = _dis_block(deg_ref)
    h1 = jnp.maximum((s_ref[0] + s_ref[1] + hp_ref[...]) * dis, 0.0)
    h = h1 * _bcos_scale(h1)
    o_ref[...] = jnp.dot(h, w_ref[...], preferred_element_type=jnp.float32,
                         precision=lax.Precision.HIGHEST) * dis


def _stage3_body(s_ref, hp_ref, deg_ref, b_ref, wc_ref, o_ref,
                 pool_acc, cnt_acc):
    i = pl.program_id(0)

    @pl.when(i == 0)
    def _():
        pool_acc[...] = jnp.zeros((NUM_GRAPHS, D), jnp.float32)
        cnt_acc[...] = jnp.zeros((NUM_GRAPHS, D), jnp.float32)

    dis = _dis_block(deg_ref)
    h2 = (s_ref[0] + s_ref[1] + hp_ref[...]) * dis
    b = b_ref[0]
    oh = (b == lax.broadcasted_iota(jnp.int32, (NUM_GRAPHS, _RB), 0))
    oh = oh.astype(jnp.float32)
    pool_acc[...] += jnp.dot(oh, h2, preferred_element_type=jnp.float32,
                             precision=lax.Precision.HIGHEST)
    cnt_acc[...] += jnp.dot(oh, jnp.ones((_RB, D), jnp.float32),
                            preferred_element_type=jnp.float32,
                            precision=lax.Precision.HIGHEST)

    @pl.when(i == pl.num_programs(0) - 1)
    def _():
        pooled = pool_acc[...] / jnp.maximum(cnt_acc[...], 1.0)
        o_ref[...] = jnp.dot(pooled, wc_ref[...],
                             preferred_element_type=jnp.float32,
                             precision=lax.Precision.HIGHEST)


_G = N_NODES // _RB

_stage1 = pl.pallas_call(
    _stage1_body,
    grid=(_G,),
    in_specs=[
        pl.BlockSpec((_RB, D), lambda i: (i, 0)),
        pl.BlockSpec((D, D), lambda i: (0, 0)),
        pl.BlockSpec((NC, _RB, DEG_W), lambda i: (0, i, 0)),
    ],
    out_specs=pl.BlockSpec((_RB, D), lambda i: (i, 0)),
    out_shape=jax.ShapeDtypeStruct((N_NODES, D), jnp.float32),
)

_stage2 = pl.pallas_call(
    _stage2_body,
    grid=(_G,),
    in_specs=[
        pl.BlockSpec((NC, _RB, D), lambda i: (0, i, 0)),
        pl.BlockSpec((_RB, D), lambda i: (i, 0)),
        pl.BlockSpec((NC, _RB, DEG_W), lambda i: (0, i, 0)),
        pl.BlockSpec((D, D), lambda i: (0, 0)),
    ],
    out_specs=pl.BlockSpec((_RB, D), lambda i: (i, 0)),
    out_shape=jax.ShapeDtypeStruct((N_NODES, D), jnp.float32),
)

_stage3 = pl.pallas_call(
    _stage3_body,
    grid=(_G,),
    in_specs=[
        pl.BlockSpec((NC, _RB, D), lambda i: (0, i, 0)),
        pl.BlockSpec((_RB, D), lambda i: (i, 0)),
        pl.BlockSpec((NC, _RB, DEG_W), lambda i: (0, i, 0)),
        pl.BlockSpec((1, 1, _RB), lambda i: (i, 0, 0)),
        pl.BlockSpec((D, NUM_CLASSES), lambda i: (0, 0)),
    ],
    out_specs=pl.BlockSpec((NUM_GRAPHS, NUM_CLASSES), lambda i: (0, 0)),
    out_shape=jax.ShapeDtypeStruct((NUM_GRAPHS, NUM_CLASSES), jnp.float32),
    scratch_shapes=[
        pltpu.VMEM((NUM_GRAPHS, D), jnp.float32),
        pltpu.VMEM((NUM_GRAPHS, D), jnp.float32),
    ],
)


def kernel(x, edge_index, batch, W1, W2, Wc):
    row3 = edge_index[0].reshape(NC, NS, C * K)
    col3 = edge_index[1].reshape(NC, NS, C, K)
    batch3 = batch.reshape(_G, 1, _RB)

    degp = _deg_kernel(col3)
    h1p = _stage1(x, W1, degp)
    s1 = _scatter_kernel(h1p, row3, col3)
    h2p = _stage2(s1, h1p, degp, W2)
    s2 = _scatter_kernel(h2p, row3, col3)
    return _stage3(s2, h2p, degp, batch3, Wc)


# speedup vs baseline: 21.8612x; 1.0081x over previous
import functools

import jax
import jax.numpy as jnp
from jax import lax
from jax.experimental import pallas as pl
from jax.experimental.pallas import tpu as pltpu
from jax.experimental.pallas import tpu_sc as plsc

N_NODES = 10000
N_EDGES = 320000
D = 128
NUM_CLASSES = 10
NUM_GRAPHS = 128
B_EXP = 1.5
EPS = 1e-8

NC = 2
NS = 16
K = 80
C = N_EDGES // (NC * NS * K)
N_PAD = 10240
ROWS_PER_TILE = N_PAD // NS
DEG_W = 16

_mesh = plsc.VectorSubcoreMesh(core_axis_name="c", subcore_axis_name="s")


@functools.partial(
    pl.kernel,
    out_type=jax.ShapeDtypeStruct((NC, N_PAD, DEG_W), jnp.float32),
    mesh=_mesh,
    scratch_types=[
        pltpu.VMEM((C, K), jnp.int32),
        pltpu.VMEM((K, DEG_W), jnp.float32),
        pltpu.VMEM((128, DEG_W), jnp.float32),
        pltpu.VMEM_SHARED((N_PAD, DEG_W), jnp.float32),
    ],
)
def _deg_kernel(col_hbm, out_hbm, col_v, ones_v, stage, accum):
    c = lax.axis_index("c")
    s = lax.axis_index("s")
    zeros16 = jnp.zeros((16,), jnp.float32)
    ones16 = jnp.ones((16,), jnp.float32)

    def _zrow(i, _):
        stage[i, :] = zeros16
        return 0

    lax.fori_loop(0, 128, _zrow, 0)

    def _orow(i, _):
        ones_v[i, :] = ones16
        return 0

    lax.fori_loop(0, K, _orow, 0)

    base = s * ROWS_PER_TILE
    for t in range(5):
        pltpu.sync_copy(stage, accum.at[pl.ds(base + t * 128, 128)])
    plsc.subcore_barrier()

    pltpu.sync_copy(col_hbm.at[c, s], col_v)

    def _body(j, _):
        pltpu.sync_copy(ones_v, accum.at[col_v.at[j]], add=True)
        return 0

    lax.fori_loop(0, C, _body, 0)
    plsc.subcore_barrier()

    for t in range(5):
        pltpu.sync_copy(accum.at[pl.ds(base + t * 128, 128)], stage)
        pltpu.sync_copy(stage, out_hbm.at[c, pl.ds(base + t * 128, 128)])


@functools.partial(
    pl.kernel,
    out_type=jax.ShapeDtypeStruct((NC, N_PAD, D), jnp.float32),
    mesh=_mesh,
    scratch_types=[
        pltpu.VMEM((C * K,), jnp.int32),
        pltpu.VMEM((C, K), jnp.int32),
        pltpu.VMEM((2, K, D), jnp.float32),
        pltpu.VMEM_SHARED((N_PAD, D), jnp.float32),
        pltpu.SemaphoreType.DMA,
        pltpu.SemaphoreType.DMA,
    ],
)
def _scatter_kernel(h_hbm, row_hbm, col_hbm, out_hbm,
                    row_v, col_v, msg, accum, sem, ssem):
    c = lax.axis_index("c")
    s = lax.axis_index("s")
    zeros16 = jnp.zeros((16,), jnp.float32)

    def _zrow(i, _):
        msg[0, i // 8, pl.ds((i % 8) * 16, 16)] = zeros16
        return 0

    lax.fori_loop(0, K * 8, _zrow, 0)

    base = s * ROWS_PER_TILE
    for t in range(ROWS_PER_TILE // K):
        pltpu.sync_copy(msg.at[0], accum.at[pl.ds(base + t * K, K)])
    plsc.subcore_barrier()

    pltpu.sync_copy(row_hbm.at[c, s], row_v)
    pltpu.sync_copy(col_hbm.at[c, s], col_v)

    def _gather_desc(j, slot):
        off = pl.multiple_of(j * K, K)
        return pltpu.make_async_copy(h_hbm.at[row_v.at[pl.ds(off, K)]],
                                     msg.at[slot], sem)

    def _gather(j, slot):
        _gather_desc(j, slot).start()

    def _scatter_wait(j, slot):
        pltpu.make_async_copy(msg.at[slot], accum.at[col_v.at[j]], ssem).wait()

    _gather(0, 0)

    def _body(j, _):
        slot = lax.rem(j, 2)
        _gather_desc(j, slot).wait()

        @pl.when(j >= 1)
        def _():
            _scatter_wait(j - 1, 1 - slot)

        @pl.when(j + 1 < C)
        def _():
            _gather(j + 1, 1 - slot)

        pltpu.async_copy(msg.at[slot], accum.at[col_v.at[j]], ssem, add=True)
        return 0

    lax.fori_loop(0, C, _body, 0)
    _scatter_wait(C - 1, (C - 1) % 2)
    plsc.subcore_barrier()

    nT = ROWS_PER_TILE // K
    for t in range(nT):
        slot = t % 2
        if t >= 2:
            pltpu.make_async_copy(
                msg.at[slot], out_hbm.at[c, pl.ds(base + (t - 2) * K, K)],
                ssem).wait()
        pltpu.sync_copy(accum.at[pl.ds(base + t * K, K)], msg.at[slot])
        pltpu.make_async_copy(
            msg.at[slot], out_hbm.at[c, pl.ds(base + t * K, K)], ssem).start()
    for t in (nT - 2, nT - 1):
        pltpu.make_async_copy(
            msg.at[t % 2], out_hbm.at[c, pl.ds(base + t * K, K)], ssem).wait()


_RB = 1000


def _dis_block(deg_ref):
    deg = deg_ref[0, :, 0:1] + deg_ref[1, :, 0:1] + 1.0
    return lax.rsqrt(deg)


def _bcos_scale(h):
    nrm2 = jnp.sum(h * h, axis=1, keepdims=True)
    return lax.rsqrt(jnp.sqrt(nrm2) + EPS)


def _stage1_body(x_ref, w_ref, deg_ref, o_ref):
    x = x_ref[...]
    dis = _dis_block(deg_ref)
    h = x * _bcos_scale(x)
    o_ref[...] = jnp.dot(h, w_ref[...], preferred_element_type=jnp.float32,
                         precision=lax.Precision.HIGHEST) * dis


def _stage2_body(s_ref, hp_ref, deg_ref, w_ref, o_ref):
    dis = _dis_block(deg_ref)
    h1 = jnp.maximum((s_ref[0] + s_ref[1] + hp_ref[...]) * dis, 0.0)
    h = h1 * _bcos_scale(h1)
    o_ref[...] = jnp.dot(h, w_ref[...], preferred_element_type=jnp.float32,
                         precision=lax.Precision.HIGHEST) * dis


def _stage3_body(s_ref, hp_ref, deg_ref, b_ref, wc_ref, o_ref,
                 pool_acc, cnt_acc):
    i = pl.program_id(0)

    @pl.when(i == 0)
    def _():
        pool_acc[...] = jnp.zeros((NUM_GRAPHS, D), jnp.float32)
        cnt_acc[...] = jnp.zeros((NUM_GRAPHS, D), jnp.float32)

    dis = _dis_block(deg_ref)
    h2 = (s_ref[0] + s_ref[1] + hp_ref[...]) * dis
    b = b_ref[0]
    oh = (b == lax.broadcasted_iota(jnp.int32, (NUM_GRAPHS, _RB), 0))
    oh = oh.astype(jnp.float32)
    pool_acc[...] += jnp.dot(oh, h2, preferred_element_type=jnp.float32,
                             precision=lax.Precision.HIGHEST)
    cnt_acc[...] += jnp.dot(oh, jnp.ones((_RB, D), jnp.float32),
                            preferred_element_type=jnp.float32,
                            precision=lax.Precision.HIGHEST)

    @pl.when(i == pl.num_programs(0) - 1)
    def _():
        pooled = pool_acc[...] / jnp.maximum(cnt_acc[...], 1.0)
        o_ref[...] = jnp.dot(pooled, wc_ref[...],
                             preferred_element_type=jnp.float32,
                             precision=lax.Precision.HIGHEST)


_G = N_NODES // _RB

_stage1 = pl.pallas_call(
    _stage1_body,
    grid=(_G,),
    in_specs=[
        pl.BlockSpec((_RB, D), lambda i: (i, 0)),
        pl.BlockSpec((D, D), lambda i: (0, 0)),
        pl.BlockSpec((NC, _RB, DEG_W), lambda i: (0, i, 0)),
    ],
    out_specs=pl.BlockSpec((_RB, D), lambda i: (i, 0)),
    out_shape=jax.ShapeDtypeStruct((N_NODES, D), jnp.float32),
)

_stage2 = pl.pallas_call(
    _stage2_body,
    grid=(_G,),
    in_specs=[
        pl.BlockSpec((NC, _RB, D), lambda i: (0, i, 0)),
        pl.BlockSpec((_RB, D), lambda i: (i, 0)),
        pl.BlockSpec((NC, _RB, DEG_W), lambda i: (0, i, 0)),
        pl.BlockSpec((D, D), lambda i: (0, 0)),
    ],
    out_specs=pl.BlockSpec((_RB, D), lambda i: (i, 0)),
    out_shape=jax.ShapeDtypeStruct((N_NODES, D), jnp.float32),
)

_stage3 = pl.pallas_call(
    _stage3_body,
    grid=(_G,),
    in_specs=[
        pl.BlockSpec((NC, _RB, D), lambda i: (0, i, 0)),
        pl.BlockSpec((_RB, D), lambda i: (i, 0)),
        pl.BlockSpec((NC, _RB, DEG_W), lambda i: (0, i, 0)),
        pl.BlockSpec((1, 1, _RB), lambda i: (i, 0, 0)),
        pl.BlockSpec((D, NUM_CLASSES), lambda i: (0, 0)),
    ],
    out_specs=pl.BlockSpec((NUM_GRAPHS, NUM_CLASSES), lambda i: (0, 0)),
    out_shape=jax.ShapeDtypeStruct((NUM_GRAPHS, NUM_CLASSES), jnp.float32),
    scratch_shapes=[
        pltpu.VMEM((NUM_GRAPHS, D), jnp.float32),
        pltpu.VMEM((NUM_GRAPHS, D), jnp.float32),
    ],
)


def kernel(x, edge_index, batch, W1, W2, Wc):
    row3 = edge_index[0].reshape(NC, NS, C * K)
    col3 = edge_index[1].reshape(NC, NS, C, K)
    batch3 = batch.reshape(_G, 1, _RB)

    degp = _deg_kernel(col3)
    h1p = _stage1(x, W1, degp)
    s1 = _scatter_kernel(h1p, row3, col3)
    h2p = _stage2(s1, h1p, degp, W2)
    s2 = _scatter_kernel(h2p, row3, col3)
    return _stage3(s2, h2p, degp, batch3, Wc)
